# Initial kernel scaffold; baseline (speedup 1.0000x reference)
import jax
import jax.numpy as jnp
from jax.experimental import pallas as pl


def kernel(x, edge_index, layer, W, b_lin, anchor):
    raise NotImplementedError("write your pallas kernel here")


# speedup vs baseline: 3.7182x; 3.7182x over previous
import functools

import jax
import jax.numpy as jnp
from jax import lax
from jax.experimental import pallas as pl
from jax.experimental.pallas import tpu as pltpu
from jax.experimental.pallas import tpu_sc as plsc

_LANES = 16
_NC = 2
_NS = 16
_NW = _NC * _NS
_IDXW = 128
_BLK = 4096


def _proj_body(n, x_ref, w_ref, bp_ref, out_ref):
    x = x_ref[...]
    p = jnp.dot(x, w_ref[...], preferred_element_type=jnp.float32)
    out_ref[0:n, :] = p[:, 0:_LANES] + bp_ref[...]
    out_ref[n : 2 * n, :] = p[:, _LANES : 2 * _LANES]


def _make_tables(x, W, b_lin):
    n, d = x.shape
    a = W.shape[1]
    ws = jnp.pad(W[:d], ((0, 0), (0, _LANES - a)))
    wd = jnp.pad(W[d:], ((0, 0), (0, _LANES - a)))
    wfull = jnp.concatenate([ws, wd], axis=1)
    bp = jnp.full((1, _LANES), -1e30, dtype=jnp.float32)
    bp = bp.at[0, :a].set(b_lin.astype(jnp.float32))
    return pl.pallas_call(
        functools.partial(_proj_body, n),
        out_shape=jax.ShapeDtypeStruct((2 * n, _LANES), jnp.float32),
    )(x.astype(jnp.float32), wfull, bp)


def _gather_add_body(edges_per_w, chunk_rows, table_ref, si_ref, di_ref,
                     out_ref, si_v, di_v, a_v, b_v, sem):
    wid = lax.axis_index("s") * _NC + lax.axis_index("c")
    e0 = wid * edges_per_w
    pltpu.sync_copy(si_ref.at[pl.ds(e0, edges_per_w)], si_v)
    pltpu.sync_copy(di_ref.at[pl.ds(e0, edges_per_w)], di_v)
    cedges = chunk_rows * _IDXW
    nchunks = edges_per_w // cedges
    for ch in range(nchunks):
        descs = []
        for j in range(chunk_rows):
            o = ch * cedges + j * _IDXW
            descs.append(pltpu.async_copy(
                table_ref.at[si_v.at[pl.ds(o, _IDXW)]],
                a_v.at[pl.ds(j * _IDXW, _IDXW)], sem))
            descs.append(pltpu.async_copy(
                table_ref.at[di_v.at[pl.ds(o, _IDXW)]],
                b_v.at[pl.ds(j * _IDXW, _IDXW)], sem))
        for dsc in descs:
            dsc.wait()

        def _add(i, _):
            a_v[i, :] = a_v[i, :] + b_v[i, :]
            return 0

        lax.fori_loop(0, cedges, _add, 0, unroll=8)
        pltpu.sync_copy(a_v, out_ref.at[pl.ds(e0 + ch * cedges, cedges)])


def _gather_add(table, si, di, ep, chunk_rows):
    edges_per_w = ep // _NW
    mesh = plsc.VectorSubcoreMesh(
        core_axis_name="c", subcore_axis_name="s",
        num_cores=_NC, num_subcores=_NS)
    cedges = chunk_rows * _IDXW
    fn = pl.kernel(
        functools.partial(_gather_add_body, edges_per_w, chunk_rows),
        out_type=jax.ShapeDtypeStruct((ep, _LANES), jnp.float32),
        mesh=mesh,
        compiler_params=pltpu.CompilerParams(use_tc_tiling_on_sc=False),
        scratch_types=[
            pltpu.VMEM((edges_per_w,), jnp.int32),
            pltpu.VMEM((edges_per_w,), jnp.int32),
            pltpu.VMEM((cedges, _LANES), jnp.float32),
            pltpu.VMEM((cedges, _LANES), jnp.float32),
            pltpu.SemaphoreType.DMA,
        ],
    )
    return fn(table, si, di)


def _attn_body(l_ref, anc_ref, out_ref):
    l = l_ref[...]
    l = jnp.where(l >= 0, l, 0.01 * l)
    m = jnp.max(l, axis=1, keepdims=True)
    e = jnp.exp(l - m)
    att = e / jnp.sum(e, axis=1, keepdims=True)
    out_ref[...] = jnp.dot(att, anc_ref[...], preferred_element_type=jnp.float32)


def _attn(logits, anchor_pad, ep, d):
    nblk = ep // _BLK
    return pl.pallas_call(
        _attn_body,
        grid=(nblk,),
        in_specs=[
            pl.BlockSpec((_BLK, _LANES), lambda i: (i, 0)),
            pl.BlockSpec((_LANES, d), lambda i: (0, 0)),
        ],
        out_specs=pl.BlockSpec((_BLK, d), lambda i: (i, 0)),
        out_shape=jax.ShapeDtypeStruct((ep, d), jnp.float32),
    )(logits, anchor_pad)


def kernel(x, edge_index, layer, W, b_lin, anchor):
    del layer
    n, d = x.shape
    a = anchor.shape[0]
    e = edge_index.shape[1]
    etot = e + n

    quantum = _NW * _IDXW
    ep = -(-etot // quantum) * quantum

    table = _make_tables(x, W, b_lin)

    sl = jnp.arange(n, dtype=jnp.int32)
    si = jnp.concatenate([edge_index[0].astype(jnp.int32), sl])
    di = jnp.concatenate([edge_index[1].astype(jnp.int32), sl]) + n
    si = jnp.pad(si, (0, ep - etot))
    di = jnp.pad(di, (0, ep - etot), constant_values=n)

    rows_per_w = (ep // _IDXW) // _NW
    chunk_rows = 1
    for c in (9, 8, 12, 6, 4, 3, 2):
        if rows_per_w % c == 0:
            chunk_rows = c
            break
    logits = _gather_add(table, si, di, ep, chunk_rows)

    anchor_pad = jnp.pad(anchor.astype(jnp.float32), ((0, _LANES - a), (0, 0)))
    out = _attn(logits, anchor_pad, ep, d)
    return out[:etot]


# speedup vs baseline: 4.6859x; 1.2603x over previous
import functools

import jax
import jax.numpy as jnp
from jax import lax
from jax.experimental import pallas as pl
from jax.experimental.pallas import tpu as pltpu
from jax.experimental.pallas import tpu_sc as plsc

_LANES = 16
_NC = 2
_NS = 16
_NW = _NC * _NS
_IDXW = 128
_BLK = 4096


def _proj_body(n, x_ref, w_ref, bp_ref, out_ref):
    x = x_ref[...]
    p = jnp.dot(x, w_ref[...], preferred_element_type=jnp.float32)
    out_ref[0:n, :] = p[:, 0:_LANES] + bp_ref[...]
    out_ref[n : 2 * n, :] = p[:, _LANES : 2 * _LANES]


def _make_tables(x, W, b_lin):
    n, d = x.shape
    a = W.shape[1]
    ws = jnp.pad(W[:d], ((0, 0), (0, _LANES - a)))
    wd = jnp.pad(W[d:], ((0, 0), (0, _LANES - a)))
    wfull = jnp.concatenate([ws, wd], axis=1)
    bp = jnp.full((1, _LANES), -1e30, dtype=jnp.float32)
    bp = bp.at[0, :a].set(b_lin.astype(jnp.float32))
    return pl.pallas_call(
        functools.partial(_proj_body, n),
        out_shape=jax.ShapeDtypeStruct((2 * n, _LANES), jnp.float32),
    )(x.astype(jnp.float32), wfull, bp)


def _gather_add_body(edges_per_w, chunk_rows, table_ref, si_ref, di_ref,
                     out_ref, si_v, di_v, a_v, b_v, sem):
    wid = lax.axis_index("s") * _NC + lax.axis_index("c")
    e0 = wid * edges_per_w
    pltpu.sync_copy(si_ref.at[pl.ds(e0, edges_per_w)], si_v)
    pltpu.sync_copy(di_ref.at[pl.ds(e0, edges_per_w)], di_v)
    cedges = chunk_rows * _IDXW
    nchunks = edges_per_w // cedges
    for ch in range(nchunks):
        descs = []
        for j in range(chunk_rows):
            o = ch * cedges + j * _IDXW
            descs.append(pltpu.async_copy(
                table_ref.at[si_v.at[pl.ds(o, _IDXW)]],
                a_v.at[pl.ds(j * _IDXW, _IDXW)], sem))
            descs.append(pltpu.async_copy(
                table_ref.at[di_v.at[pl.ds(o, _IDXW)]],
                b_v.at[pl.ds(j * _IDXW, _IDXW)], sem))
        for dsc in descs:
            dsc.wait()

        def _add(i, _):
            a_v[i, :] = a_v[i, :] + b_v[i, :]
            return 0

        lax.fori_loop(0, cedges, _add, 0, unroll=8)
        pltpu.sync_copy(a_v, out_ref.at[pl.ds(e0 + ch * cedges, cedges)])


def _gather_add(table, si, di, ep, chunk_rows):
    edges_per_w = ep // _NW
    mesh = plsc.VectorSubcoreMesh(
        core_axis_name="c", subcore_axis_name="s",
        num_cores=_NC, num_subcores=_NS)
    cedges = chunk_rows * _IDXW
    fn = pl.kernel(
        functools.partial(_gather_add_body, edges_per_w, chunk_rows),
        out_type=jax.ShapeDtypeStruct((ep, _LANES), jnp.float32),
        mesh=mesh,
        compiler_params=pltpu.CompilerParams(use_tc_tiling_on_sc=False),
        scratch_types=[
            pltpu.VMEM((edges_per_w,), jnp.int32),
            pltpu.VMEM((edges_per_w,), jnp.int32),
            pltpu.VMEM((cedges, _LANES), jnp.float32),
            pltpu.VMEM((cedges, _LANES), jnp.float32),
            pltpu.SemaphoreType.DMA,
        ],
    )
    return fn(table, si, di)


def _attn_body(l_ref, anc_ref, out_ref):
    l = l_ref[...]
    l = jnp.where(l >= 0, l, 0.01 * l)
    m = jnp.max(l, axis=1, keepdims=True)
    e = jnp.exp(l - m)
    att = e / jnp.sum(e, axis=1, keepdims=True)
    out_ref[...] = jnp.dot(att, anc_ref[...], preferred_element_type=jnp.float32)


def _attn(logits, anchor_pad, etot, ep, d):
    nblk = ep // _BLK
    return pl.pallas_call(
        _attn_body,
        grid=(nblk,),
        in_specs=[
            pl.BlockSpec((_BLK, _LANES), lambda i: (i, 0)),
            pl.BlockSpec((_LANES, d), lambda i: (0, 0)),
        ],
        out_specs=pl.BlockSpec((_BLK, d), lambda i: (i, 0)),
        out_shape=jax.ShapeDtypeStruct((etot, d), jnp.float32),
    )(logits, anchor_pad)


def kernel(x, edge_index, layer, W, b_lin, anchor):
    del layer
    n, d = x.shape
    a = anchor.shape[0]
    e = edge_index.shape[1]
    etot = e + n

    quantum = _NW * _IDXW
    ep = -(-etot // quantum) * quantum

    table = _make_tables(x, W, b_lin)

    sl = jnp.arange(n, dtype=jnp.int32)
    si = jnp.concatenate([edge_index[0].astype(jnp.int32), sl])
    di = jnp.concatenate([edge_index[1].astype(jnp.int32), sl]) + n
    si = jnp.pad(si, (0, ep - etot))
    di = jnp.pad(di, (0, ep - etot), constant_values=n)

    rows_per_w = (ep // _IDXW) // _NW
    chunk_rows = 1
    for c in (9, 8, 12, 6, 4, 3, 2):
        if rows_per_w % c == 0:
            chunk_rows = c
            break
    logits = _gather_add(table, si, di, ep, chunk_rows)

    anchor_pad = jnp.pad(anchor.astype(jnp.float32), ((0, _LANES - a), (0, 0)))
    return _attn(logits, anchor_pad, etot, ep, d)


# speedup vs baseline: 5.0380x; 1.0751x over previous
import functools

import jax
import jax.numpy as jnp
from jax import lax
from jax.experimental import pallas as pl
from jax.experimental.pallas import tpu as pltpu
from jax.experimental.pallas import tpu_sc as plsc

_LANES = 16
_NC = 2
_NS = 16
_NW = _NC * _NS
_IDXW = 128
_BLK = 4096


def _proj_body(n, x_ref, w_ref, bp_ref, out_ref):
    x = x_ref[...]
    p = jnp.dot(x, w_ref[...], preferred_element_type=jnp.float32)
    out_ref[0:n, :] = p[:, 0:_LANES] + bp_ref[...]
    out_ref[n : 2 * n, :] = p[:, _LANES : 2 * _LANES]


def _make_tables(x, W, b_lin):
    n, d = x.shape
    a = W.shape[1]
    ws = jnp.pad(W[:d], ((0, 0), (0, _LANES - a)))
    wd = jnp.pad(W[d:], ((0, 0), (0, _LANES - a)))
    wfull = jnp.concatenate([ws, wd], axis=1)
    bp = jnp.full((1, _LANES), -1e30, dtype=jnp.float32)
    bp = bp.at[0, :a].set(b_lin.astype(jnp.float32))
    return pl.pallas_call(
        functools.partial(_proj_body, n),
        out_shape=jax.ShapeDtypeStruct((2 * n, _LANES), jnp.float32),
    )(x.astype(jnp.float32), wfull, bp)


def _gather_add_body(edges_per_w, chunk_rows, table_ref, si_ref, di_ref,
                     out_ref, si_v, di_v, a0_v, b0_v, a1_v, b1_v,
                     g0_sem, g1_sem, s0_sem, s1_sem):
    wid = lax.axis_index("s") * _NC + lax.axis_index("c")
    e0 = wid * edges_per_w
    pltpu.sync_copy(si_ref.at[pl.ds(e0, edges_per_w)], si_v)
    pltpu.sync_copy(di_ref.at[pl.ds(e0, edges_per_w)], di_v)
    cedges = chunk_rows * _IDXW
    nchunks = edges_per_w // cedges
    a_bufs, b_bufs = (a0_v, a1_v), (b0_v, b1_v)
    g_sems, s_sems = (g0_sem, g1_sem), (s0_sem, s1_sem)

    def fire(ch, slot):
        ds = []
        for j in range(chunk_rows):
            o = ch * cedges + j * _IDXW
            ds.append(pltpu.async_copy(
                table_ref.at[si_v.at[pl.ds(o, _IDXW)]],
                a_bufs[slot].at[pl.ds(j * _IDXW, _IDXW)], g_sems[slot]))
            ds.append(pltpu.async_copy(
                table_ref.at[di_v.at[pl.ds(o, _IDXW)]],
                b_bufs[slot].at[pl.ds(j * _IDXW, _IDXW)], g_sems[slot]))
        return ds

    pend_g = {0: fire(0, 0)}
    pend_s = {}
    for ch in range(nchunks):
        slot = ch % 2
        nxt = 1 - slot
        if ch + 1 < nchunks:
            if nxt in pend_s:
                pend_s.pop(nxt).wait()
            pend_g[nxt] = fire(ch + 1, nxt)
        for dsc in pend_g.pop(slot):
            dsc.wait()
        a_v, b_v = a_bufs[slot], b_bufs[slot]

        def _add(i, _):
            a_v[i, :] = a_v[i, :] + b_v[i, :]
            return 0

        lax.fori_loop(0, cedges, _add, 0, unroll=8)
        pend_s[slot] = pltpu.async_copy(
            a_v, out_ref.at[pl.ds(e0 + ch * cedges, cedges)], s_sems[slot])
    for dsc in pend_s.values():
        dsc.wait()


def _gather_add(table, si, di, ep, chunk_rows):
    edges_per_w = ep // _NW
    mesh = plsc.VectorSubcoreMesh(
        core_axis_name="c", subcore_axis_name="s",
        num_cores=_NC, num_subcores=_NS)
    cedges = chunk_rows * _IDXW
    fn = pl.kernel(
        functools.partial(_gather_add_body, edges_per_w, chunk_rows),
        out_type=jax.ShapeDtypeStruct((ep, _LANES), jnp.float32),
        mesh=mesh,
        compiler_params=pltpu.CompilerParams(use_tc_tiling_on_sc=False),
        scratch_types=[
            pltpu.VMEM((edges_per_w,), jnp.int32),
            pltpu.VMEM((edges_per_w,), jnp.int32),
            pltpu.VMEM((cedges, _LANES), jnp.float32),
            pltpu.VMEM((cedges, _LANES), jnp.float32),
            pltpu.VMEM((cedges, _LANES), jnp.float32),
            pltpu.VMEM((cedges, _LANES), jnp.float32),
            pltpu.SemaphoreType.DMA,
            pltpu.SemaphoreType.DMA,
            pltpu.SemaphoreType.DMA,
            pltpu.SemaphoreType.DMA,
        ],
    )
    return fn(table, si, di)


def _attn_body(l_ref, anc_ref, out_ref):
    l = l_ref[...]
    l = jnp.where(l >= 0, l, 0.01 * l)
    m = jnp.max(l, axis=1, keepdims=True)
    e = jnp.exp(l - m)
    att = e / jnp.sum(e, axis=1, keepdims=True)
    out_ref[...] = jnp.dot(att, anc_ref[...], preferred_element_type=jnp.float32)


def _attn(logits, anchor_pad, etot, ep, d):
    nblk = ep // _BLK
    return pl.pallas_call(
        _attn_body,
        grid=(nblk,),
        in_specs=[
            pl.BlockSpec((_BLK, _LANES), lambda i: (i, 0)),
            pl.BlockSpec((_LANES, d), lambda i: (0, 0)),
        ],
        out_specs=pl.BlockSpec((_BLK, d), lambda i: (i, 0)),
        out_shape=jax.ShapeDtypeStruct((etot, d), jnp.float32),
    )(logits, anchor_pad)


def kernel(x, edge_index, layer, W, b_lin, anchor):
    del layer
    n, d = x.shape
    a = anchor.shape[0]
    e = edge_index.shape[1]
    etot = e + n

    quantum = _NW * _IDXW
    ep = -(-etot // quantum) * quantum

    table = _make_tables(x, W, b_lin)

    sl = jnp.arange(n, dtype=jnp.int32)
    si = jnp.concatenate([edge_index[0].astype(jnp.int32), sl])
    di = jnp.concatenate([edge_index[1].astype(jnp.int32), sl]) + n
    si = jnp.pad(si, (0, ep - etot))
    di = jnp.pad(di, (0, ep - etot), constant_values=n)

    rows_per_w = (ep // _IDXW) // _NW
    chunk_rows = 1
    for c in (9, 8, 12, 6, 4, 3, 2):
        if rows_per_w % c == 0:
            chunk_rows = c
            break
    logits = _gather_add(table, si, di, ep, chunk_rows)

    anchor_pad = jnp.pad(anchor.astype(jnp.float32), ((0, _LANES - a), (0, 0)))
    return _attn(logits, anchor_pad, etot, ep, d)


# speedup vs baseline: 6.7728x; 1.3444x over previous
import functools

import jax
import jax.numpy as jnp
from jax import lax
from jax.experimental import pallas as pl
from jax.experimental.pallas import tpu as pltpu
from jax.experimental.pallas import tpu_sc as plsc

_LANES = 16
_NC = 2
_NS = 16
_NW = _NC * _NS
_IDXW = 128
_BLK = 4096


def _proj_body(n, x_ref, w_ref, bp_ref, out_ref):
    x = x_ref[...]
    p = jnp.dot(x, w_ref[...], preferred_element_type=jnp.float32)
    out_ref[0:n, :] = p[:, 0:_LANES] + bp_ref[...]
    out_ref[n : 2 * n, :] = p[:, _LANES : 2 * _LANES]


def _make_tables(x, W, b_lin):
    n, d = x.shape
    a = W.shape[1]
    ws = jnp.pad(W[:d], ((0, 0), (0, _LANES - a)))
    wd = jnp.pad(W[d:], ((0, 0), (0, _LANES - a)))
    wfull = jnp.concatenate([ws, wd], axis=1)
    bp = jnp.full((1, _LANES), -1e30, dtype=jnp.float32)
    bp = bp.at[0, :a].set(b_lin.astype(jnp.float32))
    return pl.pallas_call(
        functools.partial(_proj_body, n),
        out_shape=jax.ShapeDtypeStruct((2 * n, _LANES), jnp.float32),
    )(x.astype(jnp.float32), wfull, bp)


def _gather_add_body(edges_per_w, chunk_rows, table_ref, si_ref, di_ref,
                     out_ref, si_v, di_v, a0_v, b0_v, a1_v, b1_v,
                     g0_sem, g1_sem, s0_sem, s1_sem):
    wid = lax.axis_index("s") * _NC + lax.axis_index("c")
    e0 = wid * edges_per_w
    pltpu.sync_copy(si_ref.at[pl.ds(e0, edges_per_w)], si_v)
    pltpu.sync_copy(di_ref.at[pl.ds(e0, edges_per_w)], di_v)
    cedges = chunk_rows * _IDXW
    nchunks = edges_per_w // cedges
    a_bufs, b_bufs = (a0_v, a1_v), (b0_v, b1_v)
    g_sems, s_sems = (g0_sem, g1_sem), (s0_sem, s1_sem)

    def fire(ch, slot):
        ds = []
        for j in range(chunk_rows):
            o = ch * cedges + j * _IDXW
            ds.append(pltpu.async_copy(
                table_ref.at[si_v.at[pl.ds(o, _IDXW)]],
                a_bufs[slot].at[pl.ds(j * _IDXW, _IDXW)], g_sems[slot]))
            ds.append(pltpu.async_copy(
                table_ref.at[di_v.at[pl.ds(o, _IDXW)]],
                b_bufs[slot].at[pl.ds(j * _IDXW, _IDXW)], g_sems[slot]))
        return ds

    pend_g = {0: fire(0, 0)}
    pend_s = {}
    for ch in range(nchunks):
        slot = ch % 2
        nxt = 1 - slot
        if ch + 1 < nchunks:
            if nxt in pend_s:
                pend_s.pop(nxt).wait()
            pend_g[nxt] = fire(ch + 1, nxt)
        for dsc in pend_g.pop(slot):
            dsc.wait()
        a_v, b_v = a_bufs[slot], b_bufs[slot]

        def _add(i, _):
            a_v[i, :] = a_v[i, :] + b_v[i, :]
            return 0

        lax.fori_loop(0, cedges, _add, 0, unroll=8)
        pend_s[slot] = pltpu.async_copy(
            a_v, out_ref.at[pl.ds(e0 + ch * cedges, cedges)], s_sems[slot])
    for dsc in pend_s.values():
        dsc.wait()


def _gather_add(table, si, di, ep, chunk_rows):
    edges_per_w = ep // _NW
    mesh = plsc.VectorSubcoreMesh(
        core_axis_name="c", subcore_axis_name="s",
        num_cores=_NC, num_subcores=_NS)
    cedges = chunk_rows * _IDXW
    fn = pl.kernel(
        functools.partial(_gather_add_body, edges_per_w, chunk_rows),
        out_type=jax.ShapeDtypeStruct((ep, _LANES), jnp.float32),
        mesh=mesh,
        compiler_params=pltpu.CompilerParams(use_tc_tiling_on_sc=False),
        scratch_types=[
            pltpu.VMEM((edges_per_w,), jnp.int32),
            pltpu.VMEM((edges_per_w,), jnp.int32),
            pltpu.VMEM((cedges, _LANES), jnp.float32),
            pltpu.VMEM((cedges, _LANES), jnp.float32),
            pltpu.VMEM((cedges, _LANES), jnp.float32),
            pltpu.VMEM((cedges, _LANES), jnp.float32),
            pltpu.SemaphoreType.DMA,
            pltpu.SemaphoreType.DMA,
            pltpu.SemaphoreType.DMA,
            pltpu.SemaphoreType.DMA,
        ],
    )
    return fn(table, si, di)


_PK = 128 // _LANES
_PBLK = _BLK // _PK


def _attn_body(l_ref, anc_ref, sseg_ref, out_ref):
    lp = l_ref[...]
    lp = jnp.maximum(lp, 0.01 * lp)
    lane = lax.broadcasted_iota(jnp.int32, lp.shape, 1) % _LANES
    m = lp
    for k in (1, 2, 4, 8):
        r = jnp.where(lane < _LANES - k,
                      pltpu.roll(m, 128 - k, axis=1),
                      pltpu.roll(m, _LANES - k, axis=1))
        m = jnp.maximum(m, r)
    e = jnp.exp(lp - m)
    s = jnp.dot(e, sseg_ref[...], preferred_element_type=jnp.float32)
    att = e / s
    anc = anc_ref[...]
    for k in range(_PK):
        out_ref[:, k, :] = jnp.dot(
            att[:, k * _LANES:(k + 1) * _LANES], anc,
            preferred_element_type=jnp.float32)


def _attn(logits, anchor_pad, etot, ep, d):
    nblk = ep // _BLK
    prows = etot // _PK
    lp = logits.reshape(ep // _PK, 128)
    sseg = jnp.kron(jnp.eye(_PK, dtype=jnp.float32),
                    jnp.ones((_LANES, _LANES), jnp.float32))
    out3 = pl.pallas_call(
        _attn_body,
        grid=(nblk,),
        in_specs=[
            pl.BlockSpec((_PBLK, 128), lambda i: (i, 0)),
            pl.BlockSpec((_LANES, d), lambda i: (0, 0)),
            pl.BlockSpec((128, 128), lambda i: (0, 0)),
        ],
        out_specs=pl.BlockSpec((_PBLK, _PK, d), lambda i: (i, 0, 0)),
        out_shape=jax.ShapeDtypeStruct((prows, _PK, d), jnp.float32),
    )(lp, anchor_pad, sseg)
    return out3.reshape(etot, d)


def kernel(x, edge_index, layer, W, b_lin, anchor):
    del layer
    n, d = x.shape
    a = anchor.shape[0]
    e = edge_index.shape[1]
    etot = e + n

    quantum = _NW * _IDXW
    ep = -(-etot // quantum) * quantum

    table = _make_tables(x, W, b_lin)

    sl = jnp.arange(n, dtype=jnp.int32)
    si = jnp.concatenate([edge_index[0].astype(jnp.int32), sl])
    di = jnp.concatenate([edge_index[1].astype(jnp.int32), sl]) + n
    si = jnp.pad(si, (0, ep - etot))
    di = jnp.pad(di, (0, ep - etot), constant_values=n)

    rows_per_w = (ep // _IDXW) // _NW
    chunk_rows = 1
    for c in (9, 8, 12, 6, 4, 3, 2):
        if rows_per_w % c == 0:
            chunk_rows = c
            break
    logits = _gather_add(table, si, di, ep, chunk_rows)

    anchor_pad = jnp.pad(anchor.astype(jnp.float32), ((0, _LANES - a), (0, 0)))
    return _attn(logits, anchor_pad, etot, ep, d)
